# baseline (speedup 1.0000x reference)
import functools

import jax
import jax.numpy as jnp
from jax.experimental import pallas as pl
from jax.experimental.pallas import tpu as pltpu

_T, _D, _H, _HD, _E, _K, _F = 2048, 1024, 16, 64, 8, 2, 768
_EPS = 1e-6


def _qkv_kernel(hid_ref, wln_ref, wqkv_ref, qkv_ref):
    x = hid_ref[:]
    var = jnp.mean(x * x, axis=-1, keepdims=True)
    h = x * jax.lax.rsqrt(var + _EPS) * wln_ref[:]
    qkv_ref[:] = jnp.dot(h.astype(jnp.bfloat16), wqkv_ref[:],
                         preferred_element_type=jnp.float32
                         ).astype(jnp.bfloat16)


def _attn_kernel(q_ref, k_ref, v_ref, o_ref, *, bq, bk, hpb):
    i = pl.program_id(1)
    scale = jnp.float32(1.0 / (_HD ** 0.5))
    qs = [q_ref[h] for h in range(hpb)]

    def one(h, j, st, masked):
        m, l, acc = st
        k = k_ref[h, pl.ds(j * bk, bk), :]
        s = jax.lax.dot_general(qs[h], k, (((1,), (1,)), ((), ())),
                                preferred_element_type=jnp.float32) * scale
        if masked:
            rr = jax.lax.broadcasted_iota(jnp.int32, (bq, bk), 0)
            cc = jax.lax.broadcasted_iota(jnp.int32, (bq, bk), 1)
            s = jnp.where(rr >= cc, s, -1e30)
        m_new = jnp.maximum(m, jnp.max(s, axis=-1, keepdims=True))
        p = jnp.exp(s - m_new)
        alpha = jnp.exp(m - m_new)
        l = l * alpha + jnp.sum(p, axis=-1, keepdims=True)
        vj = v_ref[h, pl.ds(j * bk, bk), :]
        acc = acc * alpha + jnp.dot(p.astype(jnp.bfloat16), vj,
                                    preferred_element_type=jnp.float32)
        return m_new, l, acc

    def init():
        return (jnp.full((bq, 1), -1e30, jnp.float32),
                jnp.zeros((bq, 1), jnp.float32),
                jnp.zeros((bq, _HD), jnp.float32))

    def body(j, sts):
        return tuple(one(h, j, sts[h], False) for h in range(hpb))

    sts = jax.lax.fori_loop(0, i, body, tuple(init() for _ in range(hpb)))
    for h in range(hpb):
        _, l, acc = one(h, i, sts[h], True)
        o_ref[h] = (acc / l).astype(o_ref.dtype)


def _wo_router_kernel(attn_ref, wo_ref, res_ref, wln_ref, wgate_ref,
                      hid2_ref, h2_ref, comb_ref):
    y = jnp.dot(attn_ref[:], wo_ref[:], preferred_element_type=jnp.float32)
    hid2 = res_ref[:] + y
    hid2_ref[:] = hid2
    var = jnp.mean(hid2 * hid2, axis=-1, keepdims=True)
    h2 = hid2 * jax.lax.rsqrt(var + _EPS) * wln_ref[:]
    h2_ref[:] = h2.astype(jnp.bfloat16)
    logits = jnp.dot(h2, wgate_ref[:], preferred_element_type=jnp.float32)
    lmax = jnp.max(logits, axis=-1, keepdims=True)
    ex = jnp.exp(logits - lmax)
    p = ex / jnp.sum(ex, axis=-1, keepdims=True)
    colid = jax.lax.broadcasted_iota(jnp.int32, p.shape, 1)
    m1 = jnp.max(p, axis=-1, keepdims=True)
    i1 = jnp.min(jnp.where(p == m1, colid, _E), axis=-1, keepdims=True)
    sel1 = colid == i1
    pm = jnp.where(sel1, -1.0, p)
    m2 = jnp.max(pm, axis=-1, keepdims=True)
    i2 = jnp.min(jnp.where(pm == m2, colid, _E), axis=-1, keepdims=True)
    sel2 = colid == i2
    ws = m1 + m2
    comb_ref[:] = (jnp.where(sel1, m1 / ws, 0.0)
                   + jnp.where(sel2, m2 / ws, 0.0))


def _moe_kernel(h2_ref, w1_ref, w2_ref, comb_ref, hid2_ref, out_ref):
    e = pl.program_id(1)

    @pl.when(e == 0)
    def _():
        out_ref[:] = hid2_ref[:]

    gu = jnp.dot(h2_ref[:], w1_ref[0], preferred_element_type=jnp.float32)
    g = gu[:, :_F]
    u = gu[:, _F:]
    act = (g * jax.nn.sigmoid(g) * u).astype(jnp.bfloat16)
    y = jnp.dot(act, w2_ref[0], preferred_element_type=jnp.float32)
    colid = jax.lax.broadcasted_iota(jnp.int32, comb_ref.shape, 1)
    we = jnp.sum(jnp.where(colid == e, comb_ref[:], 0.0), axis=-1,
                 keepdims=True)
    out_ref[:] = out_ref[:] + we * y


def kernel(hidden_states, w_pre_ln, wqkv, wo, w_post_ln, w_gate, w1, w2):
    f32, bf16 = jnp.float32, jnp.bfloat16
    bt = 512
    qkv = pl.pallas_call(
        _qkv_kernel,
        grid=(_T // bt,),
        in_specs=[
            pl.BlockSpec((bt, _D), lambda i: (i, 0)),
            pl.BlockSpec((_D,), lambda i: (0,)),
            pl.BlockSpec((_D, 3 * _D), lambda i: (0, 0)),
        ],
        out_specs=pl.BlockSpec((bt, 3 * _D), lambda i: (i, 0)),
        out_shape=jax.ShapeDtypeStruct((_T, 3 * _D), bf16),
        compiler_params=pltpu.CompilerParams(
            dimension_semantics=("parallel",)),
    )(hidden_states, w_pre_ln, wqkv.astype(bf16))

    q = qkv[:, :_D].reshape(_T, _H, _HD).transpose(1, 0, 2)
    k = qkv[:, _D:2 * _D].reshape(_T, _H, _HD).transpose(1, 0, 2)
    v = qkv[:, 2 * _D:].reshape(_T, _H, _HD).transpose(1, 0, 2)

    bq, bk, hpb = 512, 512, 4
    attn = pl.pallas_call(
        functools.partial(_attn_kernel, bq=bq, bk=bk, hpb=hpb),
        grid=(_H // hpb, _T // bq),
        in_specs=[
            pl.BlockSpec((hpb, bq, _HD), lambda h, i: (h, i, 0)),
            pl.BlockSpec((hpb, _T, _HD), lambda h, i: (h, 0, 0)),
            pl.BlockSpec((hpb, _T, _HD), lambda h, i: (h, 0, 0)),
        ],
        out_specs=pl.BlockSpec((hpb, bq, _HD), lambda h, i: (h, i, 0)),
        out_shape=jax.ShapeDtypeStruct((_H, _T, _HD), bf16),
        compiler_params=pltpu.CompilerParams(
            dimension_semantics=("parallel", "arbitrary")),
    )(q, k, v)
    attn = attn.transpose(1, 0, 2).reshape(_T, _D)

    bt2 = 512
    hid2, h2, comb = pl.pallas_call(
        _wo_router_kernel,
        grid=(_T // bt2,),
        in_specs=[
            pl.BlockSpec((bt2, _D), lambda i: (i, 0)),
            pl.BlockSpec((_D, _D), lambda i: (0, 0)),
            pl.BlockSpec((bt2, _D), lambda i: (i, 0)),
            pl.BlockSpec((_D,), lambda i: (0,)),
            pl.BlockSpec((_D, _E), lambda i: (0, 0)),
        ],
        out_specs=[
            pl.BlockSpec((bt2, _D), lambda i: (i, 0)),
            pl.BlockSpec((bt2, _D), lambda i: (i, 0)),
            pl.BlockSpec((bt2, _E), lambda i: (i, 0)),
        ],
        out_shape=[
            jax.ShapeDtypeStruct((_T, _D), f32),
            jax.ShapeDtypeStruct((_T, _D), bf16),
            jax.ShapeDtypeStruct((_T, _E), f32),
        ],
        compiler_params=pltpu.CompilerParams(
            dimension_semantics=("parallel",)),
    )(attn, wo.astype(bf16), hidden_states, w_post_ln, w_gate)

    btm = 1024
    out = pl.pallas_call(
        _moe_kernel,
        grid=(_T // btm, _E),
        in_specs=[
            pl.BlockSpec((btm, _D), lambda i, e: (i, 0)),
            pl.BlockSpec((1, _D, 2 * _F), lambda i, e: (e, 0, 0)),
            pl.BlockSpec((1, _F, _D), lambda i, e: (e, 0, 0)),
            pl.BlockSpec((btm, _E), lambda i, e: (i, 0)),
            pl.BlockSpec((btm, _D), lambda i, e: (i, 0)),
        ],
        out_specs=pl.BlockSpec((btm, _D), lambda i, e: (i, 0)),
        out_shape=jax.ShapeDtypeStruct((_T, _D), f32),
        compiler_params=pltpu.CompilerParams(
            dimension_semantics=("parallel", "arbitrary")),
    )(h2, w1.astype(bf16), w2.astype(bf16), comb, hid2)
    return out


# speedup vs baseline: 1.0030x; 1.0030x over previous
import functools

import jax
import jax.numpy as jnp
from jax.experimental import pallas as pl
from jax.experimental.pallas import tpu as pltpu

_T, _D, _H, _HD, _E, _K, _F = 2048, 1024, 16, 64, 8, 2, 768
_EPS = 1e-6


def _qkv_kernel(hid_ref, wln_ref, wqkv_ref, qkv_ref):
    x = hid_ref[:]
    var = jnp.mean(x * x, axis=-1, keepdims=True)
    h = x * jax.lax.rsqrt(var + _EPS) * wln_ref[:]
    qkv_ref[:] = jnp.dot(h.astype(jnp.bfloat16), wqkv_ref[:],
                         preferred_element_type=jnp.float32
                         ).astype(jnp.bfloat16)


def _attn_kernel(q_ref, k_ref, v_ref, o_ref, *, bq, bk, hpb):
    i = pl.program_id(1)
    scale = jnp.float32(1.0 / (_HD ** 0.5))
    qs = [q_ref[h] for h in range(hpb)]

    def one(h, j, st, masked):
        m, l, acc = st
        k = k_ref[h, pl.ds(j * bk, bk), :]
        s = jax.lax.dot_general(qs[h], k, (((1,), (1,)), ((), ())),
                                preferred_element_type=jnp.float32) * scale
        if masked:
            rr = jax.lax.broadcasted_iota(jnp.int32, (bq, bk), 0)
            cc = jax.lax.broadcasted_iota(jnp.int32, (bq, bk), 1)
            s = jnp.where(rr >= cc, s, -1e30)
        m_new = jnp.maximum(m, jnp.max(s, axis=-1, keepdims=True))
        p = jnp.exp(s - m_new)
        alpha = jnp.exp(m - m_new)
        l = l * alpha + jnp.sum(p, axis=-1, keepdims=True)
        vj = v_ref[h, pl.ds(j * bk, bk), :]
        acc = acc * alpha + jnp.dot(p.astype(jnp.bfloat16), vj,
                                    preferred_element_type=jnp.float32)
        return m_new, l, acc

    def init():
        return (jnp.full((bq, 1), -1e30, jnp.float32),
                jnp.zeros((bq, 1), jnp.float32),
                jnp.zeros((bq, _HD), jnp.float32))

    def body(j, sts):
        return tuple(one(h, j, sts[h], False) for h in range(hpb))

    sts = jax.lax.fori_loop(0, i, body, tuple(init() for _ in range(hpb)))
    for h in range(hpb):
        _, l, acc = one(h, i, sts[h], True)
        o_ref[h] = (acc / l).astype(o_ref.dtype)


def _wo_router_kernel(attn_ref, wo_ref, res_ref, wln_ref, wgate_ref,
                      hid2_ref, h2_ref, comb_ref):
    y = jnp.dot(attn_ref[:], wo_ref[:], preferred_element_type=jnp.float32)
    hid2 = res_ref[:] + y
    hid2_ref[:] = hid2
    var = jnp.mean(hid2 * hid2, axis=-1, keepdims=True)
    h2 = hid2 * jax.lax.rsqrt(var + _EPS) * wln_ref[:]
    h2_ref[:] = h2.astype(jnp.bfloat16)
    logits = jnp.dot(h2, wgate_ref[:], preferred_element_type=jnp.float32)
    lmax = jnp.max(logits, axis=-1, keepdims=True)
    ex = jnp.exp(logits - lmax)
    p = ex / jnp.sum(ex, axis=-1, keepdims=True)
    colid = jax.lax.broadcasted_iota(jnp.int32, p.shape, 1)
    m1 = jnp.max(p, axis=-1, keepdims=True)
    i1 = jnp.min(jnp.where(p == m1, colid, _E), axis=-1, keepdims=True)
    sel1 = colid == i1
    pm = jnp.where(sel1, -1.0, p)
    m2 = jnp.max(pm, axis=-1, keepdims=True)
    i2 = jnp.min(jnp.where(pm == m2, colid, _E), axis=-1, keepdims=True)
    sel2 = colid == i2
    ws = m1 + m2
    comb_ref[:] = (jnp.where(sel1, m1 / ws, 0.0)
                   + jnp.where(sel2, m2 / ws, 0.0))


def _moe_kernel(h2_ref, w1_ref, w2_ref, comb_ref, hid2_ref, out_ref):
    e = pl.program_id(1)

    @pl.when(e == 0)
    def _():
        out_ref[:] = hid2_ref[:]

    gu = jnp.dot(h2_ref[:], w1_ref[0], preferred_element_type=jnp.float32)
    g = gu[:, :_F]
    u = gu[:, _F:]
    act = (g * jax.nn.sigmoid(g) * u).astype(jnp.bfloat16)
    y = jnp.dot(act, w2_ref[0], preferred_element_type=jnp.float32)
    colid = jax.lax.broadcasted_iota(jnp.int32, comb_ref.shape, 1)
    we = jnp.sum(jnp.where(colid == e, comb_ref[:], 0.0), axis=-1,
                 keepdims=True)
    out_ref[:] = out_ref[:] + we * y


def kernel(hidden_states, w_pre_ln, wqkv, wo, w_post_ln, w_gate, w1, w2):
    f32, bf16 = jnp.float32, jnp.bfloat16
    bt = 512
    qkv = pl.pallas_call(
        _qkv_kernel,
        grid=(_T // bt,),
        in_specs=[
            pl.BlockSpec((bt, _D), lambda i: (i, 0)),
            pl.BlockSpec((_D,), lambda i: (0,)),
            pl.BlockSpec((_D, 3 * _D), lambda i: (0, 0)),
        ],
        out_specs=pl.BlockSpec((bt, 3 * _D), lambda i: (i, 0)),
        out_shape=jax.ShapeDtypeStruct((_T, 3 * _D), bf16),
        compiler_params=pltpu.CompilerParams(
            dimension_semantics=("parallel",)),
    )(hidden_states, w_pre_ln, wqkv.astype(bf16))

    q = qkv[:, :_D].reshape(_T, _H, _HD).transpose(1, 0, 2)
    k = qkv[:, _D:2 * _D].reshape(_T, _H, _HD).transpose(1, 0, 2)
    v = qkv[:, 2 * _D:].reshape(_T, _H, _HD).transpose(1, 0, 2)

    bq, bk, hpb = 512, 512, 8
    attn = pl.pallas_call(
        functools.partial(_attn_kernel, bq=bq, bk=bk, hpb=hpb),
        grid=(_H // hpb, _T // bq),
        in_specs=[
            pl.BlockSpec((hpb, bq, _HD), lambda h, i: (h, i, 0)),
            pl.BlockSpec((hpb, _T, _HD), lambda h, i: (h, 0, 0)),
            pl.BlockSpec((hpb, _T, _HD), lambda h, i: (h, 0, 0)),
        ],
        out_specs=pl.BlockSpec((hpb, bq, _HD), lambda h, i: (h, i, 0)),
        out_shape=jax.ShapeDtypeStruct((_H, _T, _HD), bf16),
        compiler_params=pltpu.CompilerParams(
            dimension_semantics=("parallel", "arbitrary")),
    )(q, k, v)
    attn = attn.transpose(1, 0, 2).reshape(_T, _D)

    bt2 = 512
    hid2, h2, comb = pl.pallas_call(
        _wo_router_kernel,
        grid=(_T // bt2,),
        in_specs=[
            pl.BlockSpec((bt2, _D), lambda i: (i, 0)),
            pl.BlockSpec((_D, _D), lambda i: (0, 0)),
            pl.BlockSpec((bt2, _D), lambda i: (i, 0)),
            pl.BlockSpec((_D,), lambda i: (0,)),
            pl.BlockSpec((_D, _E), lambda i: (0, 0)),
        ],
        out_specs=[
            pl.BlockSpec((bt2, _D), lambda i: (i, 0)),
            pl.BlockSpec((bt2, _D), lambda i: (i, 0)),
            pl.BlockSpec((bt2, _E), lambda i: (i, 0)),
        ],
        out_shape=[
            jax.ShapeDtypeStruct((_T, _D), f32),
            jax.ShapeDtypeStruct((_T, _D), bf16),
            jax.ShapeDtypeStruct((_T, _E), f32),
        ],
        compiler_params=pltpu.CompilerParams(
            dimension_semantics=("parallel",)),
    )(attn, wo.astype(bf16), hidden_states, w_post_ln, w_gate)

    btm = 1024
    out = pl.pallas_call(
        _moe_kernel,
        grid=(_T // btm, _E),
        in_specs=[
            pl.BlockSpec((btm, _D), lambda i, e: (i, 0)),
            pl.BlockSpec((1, _D, 2 * _F), lambda i, e: (e, 0, 0)),
            pl.BlockSpec((1, _F, _D), lambda i, e: (e, 0, 0)),
            pl.BlockSpec((btm, _E), lambda i, e: (i, 0)),
            pl.BlockSpec((btm, _D), lambda i, e: (i, 0)),
        ],
        out_specs=pl.BlockSpec((btm, _D), lambda i, e: (i, 0)),
        out_shape=jax.ShapeDtypeStruct((_T, _D), f32),
        compiler_params=pltpu.CompilerParams(
            dimension_semantics=("parallel", "arbitrary")),
    )(h2, w1.astype(bf16), w2.astype(bf16), comb, hid2)
    return out


# speedup vs baseline: 1.0185x; 1.0155x over previous
import functools

import jax
import jax.numpy as jnp
from jax.experimental import pallas as pl
from jax.experimental.pallas import tpu as pltpu

_T, _D, _H, _HD, _E, _K, _F = 2048, 1024, 16, 64, 8, 2, 768
_EPS = 1e-6


def _qkv_kernel(hid_ref, wln_ref, wqkv_ref, qkv_ref):
    x = hid_ref[:]
    var = jnp.mean(x * x, axis=-1, keepdims=True)
    h = x * jax.lax.rsqrt(var + _EPS) * wln_ref[:]
    qkv_ref[:] = jnp.dot(h.astype(jnp.bfloat16), wqkv_ref[:],
                         preferred_element_type=jnp.float32
                         ).astype(jnp.bfloat16)


def _attn_kernel(q_ref, k_ref, v_ref, o_ref, *, bq, bk, hpb):
    i = pl.program_id(1)
    qs, ms = [], []
    for h in range(hpb):
        q = q_ref[h]
        qf = q.astype(jnp.float32)
        qn = jnp.sqrt(jnp.sum(qf * qf, axis=-1, keepdims=True))
        kf = k_ref[h].astype(jnp.float32)
        kn = jnp.sqrt(jnp.max(jnp.sum(kf * kf, axis=-1, keepdims=True)))
        qs.append(q)
        ms.append(qn * kn + 1.0)

    def one(h, j, acc, masked):
        k = k_ref[h, pl.ds(j * bk, bk), :]
        s = jax.lax.dot_general(qs[h], k, (((1,), (1,)), ((), ())),
                                preferred_element_type=jnp.float32)
        if masked:
            rr = jax.lax.broadcasted_iota(jnp.int32, (bq, bk), 0)
            cc = jax.lax.broadcasted_iota(jnp.int32, (bq, bk), 1)
            s = jnp.where(rr >= cc, s, -jnp.inf)
        p = jnp.exp(s - ms[h])
        vj = v_ref[h, pl.ds(j * bk, bk), :]
        return acc + jnp.dot(p.astype(jnp.bfloat16), vj,
                             preferred_element_type=jnp.float32)

    def body(j, accs):
        return tuple(one(h, j, accs[h], False) for h in range(hpb))

    accs = jax.lax.fori_loop(
        0, i, body,
        tuple(jnp.zeros((bq, _HD + 1), jnp.float32) for _ in range(hpb)))
    for h in range(hpb):
        acc = one(h, i, accs[h], True)
        l = jnp.maximum(acc[:, _HD:_HD + 1], 1e-30)
        o_ref[h] = (acc[:, :_HD] / l).astype(o_ref.dtype)


def _wo_router_kernel(attn_ref, wo_ref, res_ref, wln_ref, wgate_ref,
                      hid2_ref, h2_ref, comb_ref):
    y = jnp.dot(attn_ref[:], wo_ref[:], preferred_element_type=jnp.float32)
    hid2 = res_ref[:] + y
    hid2_ref[:] = hid2
    var = jnp.mean(hid2 * hid2, axis=-1, keepdims=True)
    h2 = hid2 * jax.lax.rsqrt(var + _EPS) * wln_ref[:]
    h2_ref[:] = h2.astype(jnp.bfloat16)
    logits = jnp.dot(h2, wgate_ref[:], preferred_element_type=jnp.float32)
    lmax = jnp.max(logits, axis=-1, keepdims=True)
    ex = jnp.exp(logits - lmax)
    p = ex / jnp.sum(ex, axis=-1, keepdims=True)
    colid = jax.lax.broadcasted_iota(jnp.int32, p.shape, 1)
    m1 = jnp.max(p, axis=-1, keepdims=True)
    i1 = jnp.min(jnp.where(p == m1, colid, _E), axis=-1, keepdims=True)
    sel1 = colid == i1
    pm = jnp.where(sel1, -1.0, p)
    m2 = jnp.max(pm, axis=-1, keepdims=True)
    i2 = jnp.min(jnp.where(pm == m2, colid, _E), axis=-1, keepdims=True)
    sel2 = colid == i2
    ws = m1 + m2
    comb_ref[:] = (jnp.where(sel1, m1 / ws, 0.0)
                   + jnp.where(sel2, m2 / ws, 0.0))


def _moe_kernel(h2_ref, w1_ref, w2_ref, comb_ref, hid2_ref, out_ref):
    e = pl.program_id(1)

    @pl.when(e == 0)
    def _():
        out_ref[:] = hid2_ref[:]

    gu = jnp.dot(h2_ref[:], w1_ref[0], preferred_element_type=jnp.float32)
    g = gu[:, :_F]
    u = gu[:, _F:]
    act = (g * jax.nn.sigmoid(g) * u).astype(jnp.bfloat16)
    y = jnp.dot(act, w2_ref[0], preferred_element_type=jnp.float32)
    colid = jax.lax.broadcasted_iota(jnp.int32, comb_ref.shape, 1)
    we = jnp.sum(jnp.where(colid == e, comb_ref[:], 0.0), axis=-1,
                 keepdims=True)
    out_ref[:] = out_ref[:] + we * y


def kernel(hidden_states, w_pre_ln, wqkv, wo, w_post_ln, w_gate, w1, w2):
    f32, bf16 = jnp.float32, jnp.bfloat16
    bt = 512
    qkv = pl.pallas_call(
        _qkv_kernel,
        grid=(_T // bt,),
        in_specs=[
            pl.BlockSpec((bt, _D), lambda i: (i, 0)),
            pl.BlockSpec((_D,), lambda i: (0,)),
            pl.BlockSpec((_D, 3 * _D), lambda i: (0, 0)),
        ],
        out_specs=pl.BlockSpec((bt, 3 * _D), lambda i: (i, 0)),
        out_shape=jax.ShapeDtypeStruct((_T, 3 * _D), bf16),
        compiler_params=pltpu.CompilerParams(
            dimension_semantics=("parallel",)),
    )(hidden_states, w_pre_ln, wqkv.astype(bf16))

    q = (qkv[:, :_D] * jnp.bfloat16(0.125)).reshape(_T, _H, _HD).transpose(1, 0, 2)
    k = qkv[:, _D:2 * _D].reshape(_T, _H, _HD).transpose(1, 0, 2)
    v = qkv[:, 2 * _D:].reshape(_T, _H, _HD).transpose(1, 0, 2)
    v = jnp.concatenate([v, jnp.ones((_H, _T, 1), bf16)], axis=-1)

    bq, bk, hpb = 512, 512, 4
    attn = pl.pallas_call(
        functools.partial(_attn_kernel, bq=bq, bk=bk, hpb=hpb),
        grid=(_H // hpb, _T // bq),
        in_specs=[
            pl.BlockSpec((hpb, bq, _HD), lambda h, i: (h, i, 0)),
            pl.BlockSpec((hpb, _T, _HD), lambda h, i: (h, 0, 0)),
            pl.BlockSpec((hpb, _T, _HD + 1), lambda h, i: (h, 0, 0)),
        ],
        out_specs=pl.BlockSpec((hpb, bq, _HD), lambda h, i: (h, i, 0)),
        out_shape=jax.ShapeDtypeStruct((_H, _T, _HD), bf16),
        compiler_params=pltpu.CompilerParams(
            dimension_semantics=("parallel", "arbitrary")),
    )(q, k, v)
    attn = attn.transpose(1, 0, 2).reshape(_T, _D)

    bt2 = 512
    hid2, h2, comb = pl.pallas_call(
        _wo_router_kernel,
        grid=(_T // bt2,),
        in_specs=[
            pl.BlockSpec((bt2, _D), lambda i: (i, 0)),
            pl.BlockSpec((_D, _D), lambda i: (0, 0)),
            pl.BlockSpec((bt2, _D), lambda i: (i, 0)),
            pl.BlockSpec((_D,), lambda i: (0,)),
            pl.BlockSpec((_D, _E), lambda i: (0, 0)),
        ],
        out_specs=[
            pl.BlockSpec((bt2, _D), lambda i: (i, 0)),
            pl.BlockSpec((bt2, _D), lambda i: (i, 0)),
            pl.BlockSpec((bt2, _E), lambda i: (i, 0)),
        ],
        out_shape=[
            jax.ShapeDtypeStruct((_T, _D), f32),
            jax.ShapeDtypeStruct((_T, _D), bf16),
            jax.ShapeDtypeStruct((_T, _E), f32),
        ],
        compiler_params=pltpu.CompilerParams(
            dimension_semantics=("parallel",)),
    )(attn, wo.astype(bf16), hidden_states, w_post_ln, w_gate)

    btm = 1024
    out = pl.pallas_call(
        _moe_kernel,
        grid=(_T // btm, _E),
        in_specs=[
            pl.BlockSpec((btm, _D), lambda i, e: (i, 0)),
            pl.BlockSpec((1, _D, 2 * _F), lambda i, e: (e, 0, 0)),
            pl.BlockSpec((1, _F, _D), lambda i, e: (e, 0, 0)),
            pl.BlockSpec((btm, _E), lambda i, e: (i, 0)),
            pl.BlockSpec((btm, _D), lambda i, e: (i, 0)),
        ],
        out_specs=pl.BlockSpec((btm, _D), lambda i, e: (i, 0)),
        out_shape=jax.ShapeDtypeStruct((_T, _D), f32),
        compiler_params=pltpu.CompilerParams(
            dimension_semantics=("parallel", "arbitrary")),
    )(h2, w1.astype(bf16), w2.astype(bf16), comb, hid2)
    return out


# speedup vs baseline: 1.1336x; 1.1130x over previous
import functools

import jax
import jax.numpy as jnp
from jax.experimental import pallas as pl
from jax.experimental.pallas import tpu as pltpu

_T, _D, _H, _HD, _E, _K, _F = 2048, 1024, 16, 64, 8, 2, 768
_EPS = 1e-6


def _qkv_kernel(hid_ref, wln_ref, wqkv_ref, qkv_ref):
    x = hid_ref[:]
    var = jnp.mean(x * x, axis=-1, keepdims=True)
    h = x * jax.lax.rsqrt(var + _EPS) * wln_ref[:]
    qkv_ref[:] = jnp.dot(h.astype(jnp.bfloat16), wqkv_ref[:],
                         preferred_element_type=jnp.float32
                         ).astype(jnp.bfloat16)


def _attn_kernel(q_ref, k_ref, v_ref, o_ref, kn_scr, *, bq, bk, hpb):
    i = pl.program_id(1)

    @pl.when(i == 0)
    def _():
        for h in range(hpb):
            kf = k_ref[h].astype(jnp.float32)
            kn_scr[h] = jnp.sqrt(jnp.max(jnp.sum(kf * kf, axis=-1)))

    qs, ms = [], []
    for h in range(hpb):
        q = q_ref[h]
        qf = q.astype(jnp.float32)
        qn = jnp.sqrt(jnp.sum(qf * qf, axis=-1, keepdims=True))
        qs.append(q)
        ms.append(qn * kn_scr[h] + 1.0)

    def one(h, j, acc, masked):
        k = k_ref[h, pl.ds(j * bk, bk), :]
        s = jax.lax.dot_general(qs[h], k, (((1,), (1,)), ((), ())),
                                preferred_element_type=jnp.float32)
        if masked:
            rr = jax.lax.broadcasted_iota(jnp.int32, (bq, bk), 0)
            cc = jax.lax.broadcasted_iota(jnp.int32, (bq, bk), 1)
            s = jnp.where(rr >= cc, s, -jnp.inf)
        p = jnp.exp(s - ms[h])
        vj = v_ref[h, pl.ds(j * bk, bk), :]
        return acc + jnp.dot(p.astype(jnp.bfloat16), vj,
                             preferred_element_type=jnp.float32)

    def body(j, accs):
        return tuple(one(h, j, accs[h], False) for h in range(hpb))

    accs = jax.lax.fori_loop(
        0, i, body,
        tuple(jnp.zeros((bq, _HD + 1), jnp.float32) for _ in range(hpb)))
    for h in range(hpb):
        acc = one(h, i, accs[h], True)
        l = jnp.maximum(acc[:, _HD:_HD + 1], 1e-30)
        o_ref[h] = (acc[:, :_HD] / l).astype(o_ref.dtype)


def _wo_router_kernel(attn_ref, wo_ref, res_ref, wln_ref, wgate_ref,
                      hid2_ref, h2_ref, comb_ref):
    y = jnp.dot(attn_ref[:], wo_ref[:], preferred_element_type=jnp.float32)
    hid2 = res_ref[:] + y
    hid2_ref[:] = hid2
    var = jnp.mean(hid2 * hid2, axis=-1, keepdims=True)
    h2 = hid2 * jax.lax.rsqrt(var + _EPS) * wln_ref[:]
    h2_ref[:] = h2.astype(jnp.bfloat16)
    logits = jnp.dot(h2, wgate_ref[:], preferred_element_type=jnp.float32)
    lmax = jnp.max(logits, axis=-1, keepdims=True)
    ex = jnp.exp(logits - lmax)
    p = ex / jnp.sum(ex, axis=-1, keepdims=True)
    colid = jax.lax.broadcasted_iota(jnp.int32, p.shape, 1)
    m1 = jnp.max(p, axis=-1, keepdims=True)
    i1 = jnp.min(jnp.where(p == m1, colid, _E), axis=-1, keepdims=True)
    sel1 = colid == i1
    pm = jnp.where(sel1, -1.0, p)
    m2 = jnp.max(pm, axis=-1, keepdims=True)
    i2 = jnp.min(jnp.where(pm == m2, colid, _E), axis=-1, keepdims=True)
    sel2 = colid == i2
    ws = m1 + m2
    comb_ref[:] = (jnp.where(sel1, m1 / ws, 0.0)
                   + jnp.where(sel2, m2 / ws, 0.0))


def _moe_kernel(h2_ref, w1_ref, w2_ref, comb_ref, out_ref, *, btm, epg):
    eg = pl.program_id(0)
    e = pl.program_id(1)
    i = pl.program_id(2)

    @pl.when((e == 0) & (i == 0))
    def _():
        out_ref[:] = jnp.zeros_like(out_ref)

    w1b = w1_ref[0].astype(jnp.bfloat16)
    w2b = w2_ref[0].astype(jnp.bfloat16)
    gu = jnp.dot(h2_ref[:], w1b, preferred_element_type=jnp.float32)
    g = gu[:, :_F]
    u = gu[:, _F:]
    act = (g * jax.nn.sigmoid(g) * u).astype(jnp.bfloat16)
    y = jnp.dot(act, w2b, preferred_element_type=jnp.float32)
    eid = eg * epg + e
    colid = jax.lax.broadcasted_iota(jnp.int32, comb_ref.shape, 1)
    we = jnp.sum(jnp.where(colid == eid, comb_ref[:], 0.0), axis=-1,
                 keepdims=True)
    out_ref[0, pl.ds(i * btm, btm), :] += we * y


def kernel(hidden_states, w_pre_ln, wqkv, wo, w_post_ln, w_gate, w1, w2):
    f32, bf16 = jnp.float32, jnp.bfloat16
    bt = 512
    qkv = pl.pallas_call(
        _qkv_kernel,
        grid=(_T // bt,),
        in_specs=[
            pl.BlockSpec((bt, _D), lambda i: (i, 0)),
            pl.BlockSpec((_D,), lambda i: (0,)),
            pl.BlockSpec((_D, 3 * _D), lambda i: (0, 0)),
        ],
        out_specs=pl.BlockSpec((bt, 3 * _D), lambda i: (i, 0)),
        out_shape=jax.ShapeDtypeStruct((_T, 3 * _D), bf16),
        compiler_params=pltpu.CompilerParams(
            dimension_semantics=("parallel",)),
    )(hidden_states, w_pre_ln, wqkv.astype(bf16))

    q = (qkv[:, :_D] * jnp.bfloat16(0.125)).reshape(_T, _H, _HD).transpose(1, 0, 2)
    k = qkv[:, _D:2 * _D].reshape(_T, _H, _HD).transpose(1, 0, 2)
    v = qkv[:, 2 * _D:].reshape(_T, _H, _HD).transpose(1, 0, 2)
    v = jnp.concatenate([v, jnp.ones((_H, _T, 1), bf16)], axis=-1)

    bq, bk, hpb = 512, 512, 4
    attn = pl.pallas_call(
        functools.partial(_attn_kernel, bq=bq, bk=bk, hpb=hpb),
        grid=(_H // hpb, _T // bq),
        in_specs=[
            pl.BlockSpec((hpb, bq, _HD), lambda h, i: (h, i, 0)),
            pl.BlockSpec((hpb, _T, _HD), lambda h, i: (h, 0, 0)),
            pl.BlockSpec((hpb, _T, _HD + 1), lambda h, i: (h, 0, 0)),
        ],
        out_specs=pl.BlockSpec((hpb, bq, _HD), lambda h, i: (h, i, 0)),
        out_shape=jax.ShapeDtypeStruct((_H, _T, _HD), bf16),
        scratch_shapes=[pltpu.SMEM((hpb,), jnp.float32)],
        compiler_params=pltpu.CompilerParams(
            dimension_semantics=("parallel", "arbitrary")),
    )(q, k, v)
    attn = attn.transpose(1, 0, 2).reshape(_T, _D)

    bt2 = 512
    hid2, h2, comb = pl.pallas_call(
        _wo_router_kernel,
        grid=(_T // bt2,),
        in_specs=[
            pl.BlockSpec((bt2, _D), lambda i: (i, 0)),
            pl.BlockSpec((_D, _D), lambda i: (0, 0)),
            pl.BlockSpec((bt2, _D), lambda i: (i, 0)),
            pl.BlockSpec((_D,), lambda i: (0,)),
            pl.BlockSpec((_D, _E), lambda i: (0, 0)),
        ],
        out_specs=[
            pl.BlockSpec((bt2, _D), lambda i: (i, 0)),
            pl.BlockSpec((bt2, _D), lambda i: (i, 0)),
            pl.BlockSpec((bt2, _E), lambda i: (i, 0)),
        ],
        out_shape=[
            jax.ShapeDtypeStruct((_T, _D), f32),
            jax.ShapeDtypeStruct((_T, _D), bf16),
            jax.ShapeDtypeStruct((_T, _E), f32),
        ],
        compiler_params=pltpu.CompilerParams(
            dimension_semantics=("parallel",)),
    )(attn, wo.astype(bf16), hidden_states, w_post_ln, w_gate)

    btm = 1024
    epg = _E // 2
    partials = pl.pallas_call(
        functools.partial(_moe_kernel, btm=btm, epg=epg),
        grid=(2, epg, _T // btm),
        in_specs=[
            pl.BlockSpec((btm, _D), lambda g, e, i: (i, 0)),
            pl.BlockSpec((1, _D, 2 * _F), lambda g, e, i: (g * (_E // 2) + e, 0, 0)),
            pl.BlockSpec((1, _F, _D), lambda g, e, i: (g * (_E // 2) + e, 0, 0)),
            pl.BlockSpec((btm, _E), lambda g, e, i: (i, 0)),
        ],
        out_specs=pl.BlockSpec((1, _T, _D), lambda g, e, i: (g, 0, 0)),
        out_shape=jax.ShapeDtypeStruct((2, _T, _D), f32),
        compiler_params=pltpu.CompilerParams(
            dimension_semantics=("parallel", "arbitrary", "arbitrary")),
    )(h2, w1, w2, comb)
    return hid2 + partials[0] + partials[1]


# speedup vs baseline: 1.1724x; 1.0342x over previous
import functools

import jax
import jax.numpy as jnp
from jax.experimental import pallas as pl
from jax.experimental.pallas import tpu as pltpu

_T, _D, _H, _HD, _E, _K, _F = 2048, 1024, 16, 64, 8, 2, 768
_EPS = 1e-6


def _qkv_kernel(hid_ref, wln_ref, wqkv_ref, qkv_ref):
    x = hid_ref[:]
    var = jnp.mean(x * x, axis=-1, keepdims=True)
    h = x * jax.lax.rsqrt(var + _EPS) * wln_ref[:]
    qkv_ref[:] = jnp.dot(h.astype(jnp.bfloat16),
                         wqkv_ref[:].astype(jnp.bfloat16),
                         preferred_element_type=jnp.float32
                         ).astype(jnp.bfloat16)


def _attn_kernel(q_ref, k_ref, v_ref, o_ref, kn_scr, *, bq, bk, hpb):
    i = pl.program_id(1)

    @pl.when(i == 0)
    def _():
        for h in range(hpb):
            kf = k_ref[h].astype(jnp.float32)
            kn_scr[h] = jnp.sqrt(jnp.max(jnp.sum(kf * kf, axis=-1)))

    qs, ms = [], []
    for h in range(hpb):
        q = q_ref[h]
        qf = q.astype(jnp.float32)
        qn = jnp.sqrt(jnp.sum(qf * qf, axis=-1, keepdims=True))
        qs.append(q)
        ms.append(qn * kn_scr[h] + 1.0)

    def one(h, j, acc, masked):
        k = k_ref[h, pl.ds(j * bk, bk), :]
        s = jax.lax.dot_general(qs[h], k, (((1,), (1,)), ((), ())),
                                preferred_element_type=jnp.float32)
        if masked:
            rr = jax.lax.broadcasted_iota(jnp.int32, (bq, bk), 0)
            cc = jax.lax.broadcasted_iota(jnp.int32, (bq, bk), 1)
            s = jnp.where(rr >= cc, s, -jnp.inf)
        p = jnp.exp(s - ms[h])
        vj = v_ref[h, pl.ds(j * bk, bk), :]
        return acc + jnp.dot(p.astype(jnp.bfloat16), vj,
                             preferred_element_type=jnp.float32)

    def body(j, accs):
        return tuple(one(h, j, accs[h], False) for h in range(hpb))

    accs = jax.lax.fori_loop(
        0, i, body,
        tuple(jnp.zeros((bq, _HD + 1), jnp.float32) for _ in range(hpb)))
    for h in range(hpb):
        acc = one(h, i, accs[h], True)
        l = jnp.maximum(acc[:, _HD:_HD + 1], 1e-30)
        o_ref[h] = (acc[:, :_HD] / l).astype(o_ref.dtype)


def _wo_router_kernel(attn_ref, wo_ref, res_ref, wln_ref, wgate_ref,
                      hid2_ref, h2_ref, comb_ref):
    y = jnp.dot(attn_ref[:], wo_ref[:].astype(jnp.bfloat16),
                preferred_element_type=jnp.float32)
    hid2 = res_ref[:] + y
    hid2_ref[:] = hid2
    var = jnp.mean(hid2 * hid2, axis=-1, keepdims=True)
    h2 = hid2 * jax.lax.rsqrt(var + _EPS) * wln_ref[:]
    h2_ref[:] = h2.astype(jnp.bfloat16)
    logits = jnp.dot(h2, wgate_ref[:], preferred_element_type=jnp.float32)
    lmax = jnp.max(logits, axis=-1, keepdims=True)
    ex = jnp.exp(logits - lmax)
    p = ex / jnp.sum(ex, axis=-1, keepdims=True)
    colid = jax.lax.broadcasted_iota(jnp.int32, p.shape, 1)
    m1 = jnp.max(p, axis=-1, keepdims=True)
    i1 = jnp.min(jnp.where(p == m1, colid, _E), axis=-1, keepdims=True)
    sel1 = colid == i1
    pm = jnp.where(sel1, -1.0, p)
    m2 = jnp.max(pm, axis=-1, keepdims=True)
    i2 = jnp.min(jnp.where(pm == m2, colid, _E), axis=-1, keepdims=True)
    sel2 = colid == i2
    ws = m1 + m2
    comb_ref[:] = (jnp.where(sel1, m1 / ws, 0.0)
                   + jnp.where(sel2, m2 / ws, 0.0))


def _moe_kernel(h2_ref, w1_ref, w2_ref, comb_ref, out_ref, *, btm, epg):
    eg = pl.program_id(0)
    e = pl.program_id(1)
    i = pl.program_id(2)

    @pl.when((e == 0) & (i == 0))
    def _():
        out_ref[:] = jnp.zeros_like(out_ref)

    w1b = w1_ref[0].astype(jnp.bfloat16)
    w2b = w2_ref[0].astype(jnp.bfloat16)
    gu = jnp.dot(h2_ref[:], w1b, preferred_element_type=jnp.float32)
    g = gu[:, :_F]
    u = gu[:, _F:]
    act = (g * jax.nn.sigmoid(g) * u).astype(jnp.bfloat16)
    y = jnp.dot(act, w2b, preferred_element_type=jnp.float32)
    eid = eg * epg + e
    colid = jax.lax.broadcasted_iota(jnp.int32, comb_ref.shape, 1)
    we = jnp.sum(jnp.where(colid == eid, comb_ref[:], 0.0), axis=-1,
                 keepdims=True)
    out_ref[0, pl.ds(i * btm, btm), :] += we * y


def kernel(hidden_states, w_pre_ln, wqkv, wo, w_post_ln, w_gate, w1, w2):
    f32, bf16 = jnp.float32, jnp.bfloat16
    bt = 512
    qkv = pl.pallas_call(
        _qkv_kernel,
        grid=(_T // bt,),
        in_specs=[
            pl.BlockSpec((bt, _D), lambda i: (i, 0)),
            pl.BlockSpec((_D,), lambda i: (0,)),
            pl.BlockSpec((_D, 3 * _D), lambda i: (0, 0)),
        ],
        out_specs=pl.BlockSpec((bt, 3 * _D), lambda i: (i, 0)),
        out_shape=jax.ShapeDtypeStruct((_T, 3 * _D), bf16),
        compiler_params=pltpu.CompilerParams(
            dimension_semantics=("parallel",)),
    )(hidden_states, w_pre_ln, wqkv)

    q = (qkv[:, :_D] * jnp.bfloat16(0.125)).reshape(_T, _H, _HD).transpose(1, 0, 2)
    k = qkv[:, _D:2 * _D].reshape(_T, _H, _HD).transpose(1, 0, 2)
    v = qkv[:, 2 * _D:].reshape(_T, _H, _HD).transpose(1, 0, 2)
    v = jnp.concatenate([v, jnp.ones((_H, _T, 1), bf16)], axis=-1)

    bq, bk, hpb = 512, 512, 4
    attn = pl.pallas_call(
        functools.partial(_attn_kernel, bq=bq, bk=bk, hpb=hpb),
        grid=(_H // hpb, _T // bq),
        in_specs=[
            pl.BlockSpec((hpb, bq, _HD), lambda h, i: (h, i, 0)),
            pl.BlockSpec((hpb, _T, _HD), lambda h, i: (h, 0, 0)),
            pl.BlockSpec((hpb, _T, _HD + 1), lambda h, i: (h, 0, 0)),
        ],
        out_specs=pl.BlockSpec((hpb, bq, _HD), lambda h, i: (h, i, 0)),
        out_shape=jax.ShapeDtypeStruct((_H, _T, _HD), bf16),
        scratch_shapes=[pltpu.SMEM((hpb,), jnp.float32)],
        compiler_params=pltpu.CompilerParams(
            dimension_semantics=("parallel", "arbitrary")),
    )(q, k, v)
    attn = attn.transpose(1, 0, 2).reshape(_T, _D)

    bt2 = 512
    hid2, h2, comb = pl.pallas_call(
        _wo_router_kernel,
        grid=(_T // bt2,),
        in_specs=[
            pl.BlockSpec((bt2, _D), lambda i: (i, 0)),
            pl.BlockSpec((_D, _D), lambda i: (0, 0)),
            pl.BlockSpec((bt2, _D), lambda i: (i, 0)),
            pl.BlockSpec((_D,), lambda i: (0,)),
            pl.BlockSpec((_D, _E), lambda i: (0, 0)),
        ],
        out_specs=[
            pl.BlockSpec((bt2, _D), lambda i: (i, 0)),
            pl.BlockSpec((bt2, _D), lambda i: (i, 0)),
            pl.BlockSpec((bt2, _E), lambda i: (i, 0)),
        ],
        out_shape=[
            jax.ShapeDtypeStruct((_T, _D), f32),
            jax.ShapeDtypeStruct((_T, _D), bf16),
            jax.ShapeDtypeStruct((_T, _E), f32),
        ],
        compiler_params=pltpu.CompilerParams(
            dimension_semantics=("parallel",)),
    )(attn, wo, hidden_states, w_post_ln, w_gate)

    btm = 1024
    epg = _E // 2
    partials = pl.pallas_call(
        functools.partial(_moe_kernel, btm=btm, epg=epg),
        grid=(2, epg, _T // btm),
        in_specs=[
            pl.BlockSpec((btm, _D), lambda g, e, i: (i, 0)),
            pl.BlockSpec((1, _D, 2 * _F), lambda g, e, i: (g * (_E // 2) + e, 0, 0)),
            pl.BlockSpec((1, _F, _D), lambda g, e, i: (g * (_E // 2) + e, 0, 0)),
            pl.BlockSpec((btm, _E), lambda g, e, i: (i, 0)),
        ],
        out_specs=pl.BlockSpec((1, _T, _D), lambda g, e, i: (g, 0, 0)),
        out_shape=jax.ShapeDtypeStruct((2, _T, _D), f32),
        compiler_params=pltpu.CompilerParams(
            dimension_semantics=("parallel", "arbitrary", "arbitrary")),
    )(h2, w1, w2, comb)
    return hid2 + partials[0] + partials[1]
